# baseline (speedup 1.0000x reference)
import functools

import jax
import jax.numpy as jnp
from jax import lax
from jax.experimental import pallas as pl
from jax.experimental.pallas import tpu as pltpu
from jax.experimental.pallas import tpu_sc as plsc

N = 100000
NPAD = 100352
D = 128
E = 50000
EPAD = 50176
NREL = 14
EALL = NREL * EPAD + 2048
EH = EALL // 2
TGATHER = EH // 32
GB = 86
SCHUNK = 4096
NCHUNK = EH // SCHUNK
F32 = jnp.float32
I32 = jnp.int32

_mesh = plsc.VectorSubcoreMesh(core_axis_name="c", subcore_axis_name="s")


def _wid():
    return lax.axis_index("s") * 2 + lax.axis_index("c")


def _gather_body(feat_hbm, v_hbm, g_hbm, idx2, rbuf, gsem, wsem):
    w = _wid()
    base = w * TGATHER
    pltpu.sync_copy(v_hbm.at[w], idx2)

    def start_gather(k, p):
        pltpu.async_copy(feat_hbm.at[idx2.at[k]], rbuf.at[p], gsem.at[p])

    def wait_gather(k, p):
        pltpu.make_async_copy(feat_hbm.at[idx2.at[k]], rbuf.at[p],
                              gsem.at[p]).wait()

    def out_slice(k):
        return g_hbm.at[pl.ds(base + k * 128, 128), :]

    start_gather(0, 0)

    def step(k, carry):
        p = lax.rem(k, 2)
        q = lax.rem(k + 1, 2)
        wait_gather(k, p)

        @pl.when(k + 1 < GB)
        def _():
            @pl.when(k + 1 >= 2)
            def _():
                pltpu.make_async_copy(rbuf.at[q], out_slice(k - 1),
                                      wsem.at[q]).wait()
            start_gather(k + 1, q)

        pltpu.async_copy(rbuf.at[p], out_slice(k), wsem.at[p])
        return carry

    lax.fori_loop(0, GB, step, 0)
    pltpu.make_async_copy(rbuf.at[0], out_slice(GB - 2), wsem.at[0]).wait()
    pltpu.make_async_copy(rbuf.at[1], out_slice(GB - 1), wsem.at[1]).wait()


_gather = pl.kernel(
    _gather_body,
    out_type=jax.ShapeDtypeStruct((EH, D), F32),
    mesh=_mesh,
    compiler_params=pltpu.CompilerParams(needs_layout_passes=False),
    scratch_types=[
        pltpu.VMEM((GB, 128), I32),
        pltpu.VMEM((2, 128, D), F32),
        pltpu.SemaphoreType.DMA((2,)),
        pltpu.SemaphoreType.DMA((2,)),
    ],
)


def _scatter_body(mt_hbm, u_hbm, agg_hbm, ubuf_a, dbuf_a, ubuf_b, dbuf_b,
                  acc, usem_a, dsem_a, usem_b, dsem_b):
    w = _wid()

    def u_slice(c):
        return u_hbm.at[pl.ds(c * SCHUNK, SCHUNK)]

    def d_slice(col, c):
        return mt_hbm.at[col, pl.ds(c * SCHUNK, SCHUNK)]

    def start(col, c, ub, db, us, ds_):
        pltpu.async_copy(u_slice(c), ub, us)
        pltpu.async_copy(d_slice(col, c), db, ds_)

    def wait(col, c, ub, db, us, ds_):
        pltpu.make_async_copy(u_slice(c), ub, us).wait()
        pltpu.make_async_copy(d_slice(col, c), db, ds_).wait()

    def compute(ubuf, dbuf):
        def grp(g, mx_c):
            pk = ubuf[pl.ds(g * 16, 16)]
            d16 = dbuf[pl.ds(g * 16, 16)]
            u16 = lax.bitwise_and(pk, 0xFFFFF)
            od = lax.shift_right_logical(pk, 20)
            plsc.addupdate_scatter(acc, [u16], d16, mask=od == 0)
            return jnp.maximum(mx_c, od)

        mx = plsc.parallel_loop(0, SCHUNK // 16, unroll=32,
                                carry=jnp.zeros((16,), I32))(grp)
        mmax = jnp.max(mx)

        def extra(r, carry3):
            def grp2(g, c3):
                pk = ubuf[pl.ds(g * 16, 16)]
                d16 = dbuf[pl.ds(g * 16, 16)]
                u16 = lax.bitwise_and(pk, 0xFFFFF)
                od = lax.shift_right_logical(pk, 20)
                plsc.addupdate_scatter(acc, [u16], d16, mask=od == r)
                return c3

            return lax.fori_loop(0, SCHUNK // 16, grp2, carry3)

        lax.fori_loop(1, mmax + 1, extra, 0)

    def col_round(ri, carry):
        col = w + 32 * ri

        def zstep(i):
            acc[pl.ds(i * 16, 16)] = jnp.zeros((16,), F32)

        plsc.parallel_loop(0, NPAD // 16, unroll=16)(zstep)

        start(col, 0, ubuf_a, dbuf_a, usem_a, dsem_a)

        def pair(t, carry2):
            c0 = 2 * t
            c1 = 2 * t + 1
            wait(col, c0, ubuf_a, dbuf_a, usem_a, dsem_a)
            start(col, c1, ubuf_b, dbuf_b, usem_b, dsem_b)
            compute(ubuf_a, dbuf_a)
            wait(col, c1, ubuf_b, dbuf_b, usem_b, dsem_b)

            @pl.when(c1 + 1 < NCHUNK)
            def _():
                start(col, c1 + 1, ubuf_a, dbuf_a, usem_a, dsem_a)

            compute(ubuf_b, dbuf_b)
            return carry2

        lax.fori_loop(0, NCHUNK // 2, pair, 0)
        pltpu.sync_copy(acc, agg_hbm.at[col])
        return carry

    lax.fori_loop(0, 4, col_round, 0)


_scatter = pl.kernel(
    _scatter_body,
    out_type=jax.ShapeDtypeStruct((D, NPAD), F32),
    mesh=_mesh,
    compiler_params=pltpu.CompilerParams(needs_layout_passes=False),
    scratch_types=[
        pltpu.VMEM((SCHUNK,), I32),
        pltpu.VMEM((SCHUNK,), F32),
        pltpu.VMEM((SCHUNK,), I32),
        pltpu.VMEM((SCHUNK,), F32),
        pltpu.VMEM((NPAD,), F32),
        pltpu.SemaphoreType.DMA,
        pltpu.SemaphoreType.DMA,
        pltpu.SemaphoreType.DMA,
        pltpu.SemaphoreType.DMA,
    ],
)


def _k2_body(x_ref, w_ref, o_ref):
    b = pl.program_id(0)
    x = x_ref[...].astype(jnp.bfloat16)
    w = w_ref[0].astype(jnp.bfloat16)
    y = lax.dot_general(w, x, (((0,), (1,)), ((), ())),
                        preferred_element_type=F32)
    rel_j = lax.rem(b, 98)
    valid = jnp.where(b >= 7 * 98, 0, jnp.where(rel_j == 97, 336, 512))
    cols = lax.broadcasted_iota(I32, (D, 512), 1)
    o_ref[...] = jnp.where(cols < valid, y, 0.0)


def _k2_call(g, w7):
    nb = EH // 512
    return pl.pallas_call(
        _k2_body,
        grid=(nb,),
        in_specs=[
            pl.BlockSpec((512, D), lambda b: (b, 0)),
            pl.BlockSpec((1, D, D), lambda b: (jnp.minimum(b // 98, 6), 0, 0)),
        ],
        out_specs=pl.BlockSpec((D, 512), lambda b: (0, b)),
        out_shape=jax.ShapeDtypeStruct((D, EH), F32),
    )(g, w7)


def _k0_body(c_ref, f_ref, iw1, ib1, iw2, ib2, sw1, sb1, sw2, sb2, o_ref, ob_ref):
    h = jax.nn.relu(jnp.dot(c_ref[...], iw1[...], preferred_element_type=F32) + ib1[...])
    a = jax.nn.relu(jnp.dot(h, iw2[...], preferred_element_type=F32) + ib2[...])
    h = jax.nn.relu(jnp.dot(f_ref[...], sw1[...], preferred_element_type=F32) + sb1[...])
    b2 = jax.nn.relu(jnp.dot(h, sw2[...], preferred_element_type=F32) + sb2[...])
    out = jax.nn.relu(a + b2)
    o_ref[...] = out
    ob_ref[...] = out.astype(jnp.bfloat16)


def _k0_call(ctrs_p, feats_p, p):
    full = lambda shp: pl.BlockSpec(shp, lambda j: tuple(0 for _ in shp))
    return pl.pallas_call(
        _k0_body,
        grid=(NPAD // 1024,),
        in_specs=[
            pl.BlockSpec((1024, 2), lambda j: (j, 0)),
            pl.BlockSpec((1024, 2), lambda j: (j, 0)),
            full((2, D)), full((1, D)), full((D, D)), full((1, D)),
            full((2, D)), full((1, D)), full((D, D)), full((1, D)),
        ],
        out_specs=[pl.BlockSpec((1024, D), lambda j: (j, 0)),
                   pl.BlockSpec((1024, D), lambda j: (j, 0))],
        out_shape=[jax.ShapeDtypeStruct((NPAD, D), F32),
                   jax.ShapeDtypeStruct((NPAD, D), jnp.bfloat16)],
    )(ctrs_p, feats_p,
      p["in_W1"], p["in_b1"].reshape(1, D), p["in_W2"], p["in_b2"].reshape(1, D),
      p["seg_W1"], p["seg_b1"].reshape(1, D), p["seg_W2"], p["seg_b2"].reshape(1, D))


def _k4_body(f_ref, at_ref, at2_ref, cw, g_ref, b_ref, w1, b1, w2, b2, o_ref, ob_ref):
    feat = f_ref[...]
    ats = at_ref[...] + at2_ref[...]
    parts = [jnp.transpose(ats[:, t * 128:(t + 1) * 128]) for t in range(8)]
    agg = jnp.concatenate(parts, axis=0)
    temp = jnp.dot(feat, cw[...], preferred_element_type=F32) + agg
    mu = jnp.mean(temp, axis=-1, keepdims=True)
    var = jnp.mean((temp - mu) ** 2, axis=-1, keepdims=True)
    x = (temp - mu) / jnp.sqrt(var + 1e-5) * g_ref[...] + b_ref[...]
    x = jax.nn.relu(x)
    h = jax.nn.relu(jnp.dot(x, w1[...], preferred_element_type=F32) + b1[...])
    o = jax.nn.relu(jnp.dot(h, w2[...], preferred_element_type=F32) + b2[...])
    out = jax.nn.relu(o + feat)
    o_ref[...] = out
    ob_ref[...] = out.astype(jnp.bfloat16)


def _k4_call(feat, agg_t, agg_t2, cw, g, b, w1, b1, w2, b2):
    full = lambda shp: pl.BlockSpec(shp, lambda j: tuple(0 for _ in shp))
    return pl.pallas_call(
        _k4_body,
        grid=(NPAD // 1024,),
        in_specs=[
            pl.BlockSpec((1024, D), lambda j: (j, 0)),
            pl.BlockSpec((D, 1024), lambda j: (0, j)),
            pl.BlockSpec((D, 1024), lambda j: (0, j)),
            full((D, D)), full((1, D)), full((1, D)),
            full((D, D)), full((1, D)), full((D, D)), full((1, D)),
        ],
        out_specs=[pl.BlockSpec((1024, D), lambda j: (j, 0)),
                   pl.BlockSpec((1024, D), lambda j: (j, 0))],
        out_shape=[jax.ShapeDtypeStruct((NPAD, D), F32),
                   jax.ShapeDtypeStruct((NPAD, D), jnp.bfloat16)],
    )(feat, agg_t, agg_t2, cw, g.reshape(1, D), b.reshape(1, D),
      w1, b1.reshape(1, D), w2, b2.reshape(1, D))


def _k5_body(f_ref, m_ref, emb, wf, wm, b1, w2, b2, o_ref):
    m = m_ref[0, 0, :]
    iota = lax.broadcasted_iota(I32, (1000, 32), 1)
    oh = (m[:, None] == iota).astype(F32)
    meta = jnp.dot(oh, emb[...], preferred_element_type=F32)
    h = jax.nn.relu(jnp.dot(f_ref[...], wf[...], preferred_element_type=F32)
                    + jnp.dot(meta, wm[...], preferred_element_type=F32) + b1[...])
    o_ref[...] = jax.nn.relu(jnp.dot(h, w2[...], preferred_element_type=F32) + b2[...])


def _k5_call(feat, mark3d, emb_p, wf, wm, b1, w2, b2):
    full = lambda shp: pl.BlockSpec(shp, lambda j: tuple(0 for _ in shp))
    return pl.pallas_call(
        _k5_body,
        grid=(N // 1000,),
        in_specs=[
            pl.BlockSpec((1000, D), lambda j: (j, 0)),
            pl.BlockSpec((1, 1, 1000), lambda j: (j, 0, 0)),
            full((32, D)), full((D, D)), full((D, D)), full((1, D)),
            full((D, D)), full((1, D)),
        ],
        out_specs=pl.BlockSpec((1000, D), lambda j: (j, 0)),
        out_shape=jax.ShapeDtypeStruct((N, D), F32),
    )(feat, mark3d, emb_p, wf, wm, b1.reshape(1, D), w2, b2.reshape(1, D))


def kernel(ctrs, feats, mark_type, idcs, pre_u, pre_v, suc_u, suc_v,
           left_u, left_v, right_u, right_v, params):
    p = params
    depth = p["ctr_W"].shape[0]

    def pad_rel(a):
        pad = (jnp.arange(EPAD - E, dtype=I32) * 571) % N
        return jnp.concatenate([a.astype(I32), pad])

    def pad_rel_v(a):
        return jnp.concatenate([a.astype(I32), jnp.zeros((EPAD - E,), I32)])

    extra_u = (jnp.arange(1024, dtype=I32) * 49) % N
    u_parts = [pad_rel(pre_u[s]) for s in range(6)]
    u_parts += [pad_rel(suc_u[0]), extra_u]
    u_parts += [pad_rel(suc_u[s]) for s in range(1, 6)]
    u_parts += [pad_rel(left_u), pad_rel(right_u), extra_u]
    u_all = jnp.concatenate(u_parts)
    u2 = u_all.reshape(-1, 16)
    eq = u2[:, :, None] == u2[:, None, :]
    tril = (jnp.arange(16)[:, None] > jnp.arange(16)[None, :])
    ordi = jnp.sum(eq & tril[None], axis=-1, dtype=I32)
    u_all = (u_all | (ordi.reshape(-1) << 20)).reshape(2, EH)

    extra_v = jnp.zeros((1024,), I32)
    v_parts = [pad_rel_v(pre_v[s]) for s in range(6)]
    v_parts += [pad_rel_v(suc_v[0]), extra_v]
    v_parts += [pad_rel_v(suc_v[s]) for s in range(1, 6)]
    v_parts += [pad_rel_v(left_v), pad_rel_v(right_v), extra_v]
    v_all = jnp.concatenate(v_parts).reshape(2, 32, GB, 128)

    w_all = jnp.concatenate(
        [p["pre_W"], p["suc_W"], p["left_W"][:, None], p["right_W"][:, None]],
        axis=1)

    ctrs_p = jnp.pad(ctrs, ((0, NPAD - N), (0, 0)))
    feats_p = jnp.pad(feats, ((0, NPAD - N), (0, 0)))
    emb_p = jnp.pad(p["emb"], ((0, 32 - p["emb"].shape[0]), (0, 0)))
    mark3d = mark_type.astype(I32).reshape(N // 1000, 1, 1000)
    wf = p["meta_W1"][:D]
    wm = p["meta_W1"][D:]

    feat, featb = _k0_call(ctrs_p, feats_p, p)
    del featb
    for i in range(depth):
        g0 = _gather(feat, v_all[0])
        m0 = _k2_call(g0, w_all[i, :7])
        g1 = _gather(feat, v_all[1])
        a0 = _scatter(m0, u_all[0])
        m1 = _k2_call(g1, w_all[i, 7:])
        a1 = _scatter(m1, u_all[1])
        feat, _fb = _k4_call(feat, a0, a1, p["ctr_W"][i], p["gn_g"][i],
                             p["gn_b"][i], p["ctr2_W1"][i], p["ctr2_b1"][i],
                             p["ctr2_W2"][i], p["ctr2_b2"][i])
    out = _k5_call(feat, mark3d, emb_p, wf, wm,
                   p["meta_b1"], p["meta_W2"], p["meta_b2"])
    return (out, idcs, ctrs)


# speedup vs baseline: 1.0247x; 1.0247x over previous
import jax
import jax.numpy as jnp
from jax import lax
from jax.experimental import pallas as pl
from jax.experimental.pallas import tpu as pltpu
from jax.experimental.pallas import tpu_sc as plsc

N = 100000
NPAD = 100352
D = 128
E = 50000
EPAD = 50176
NREL = 14
EALL = NREL * EPAD + 2048
EH = EALL // 2
TGATHER = EH // 32
GB = 86
SCHUNK = 4096
NCHUNK = EH // SCHUNK
F32 = jnp.float32
I32 = jnp.int32

_mesh = plsc.VectorSubcoreMesh(core_axis_name="c", subcore_axis_name="s")


def _wid():
    return lax.axis_index("s") * 2 + lax.axis_index("c")


def _gather_body(feat_hbm, v_hbm, g_hbm, idx2, rbuf, gsem, wsem):
    w = _wid()
    base = w * TGATHER
    pltpu.sync_copy(v_hbm.at[w], idx2)

    def start_gather(k, p):
        pltpu.async_copy(feat_hbm.at[idx2.at[k]], rbuf.at[p], gsem.at[p])

    def wait_gather(k, p):
        pltpu.make_async_copy(feat_hbm.at[idx2.at[k]], rbuf.at[p],
                              gsem.at[p]).wait()

    def out_slice(k):
        return g_hbm.at[pl.ds(base + k * 128, 128), :]

    start_gather(0, 0)

    def step(k, carry):
        p = lax.rem(k, 2)
        q = lax.rem(k + 1, 2)
        wait_gather(k, p)

        @pl.when(k + 1 < GB)
        def _():
            @pl.when(k + 1 >= 2)
            def _():
                pltpu.make_async_copy(rbuf.at[q], out_slice(k - 1),
                                      wsem.at[q]).wait()
            start_gather(k + 1, q)

        pltpu.async_copy(rbuf.at[p], out_slice(k), wsem.at[p])
        return carry

    lax.fori_loop(0, GB, step, 0)
    pltpu.make_async_copy(rbuf.at[0], out_slice(GB - 2), wsem.at[0]).wait()
    pltpu.make_async_copy(rbuf.at[1], out_slice(GB - 1), wsem.at[1]).wait()


_gather = pl.kernel(
    _gather_body,
    out_type=jax.ShapeDtypeStruct((EH, D), F32),
    mesh=_mesh,
    compiler_params=pltpu.CompilerParams(needs_layout_passes=False),
    scratch_types=[
        pltpu.VMEM((GB, 128), I32),
        pltpu.VMEM((2, 128, D), F32),
        pltpu.SemaphoreType.DMA((2,)),
        pltpu.SemaphoreType.DMA((2,)),
    ],
)


def _scatter_body(mt_hbm, u_hbm, agg_hbm, ubuf_a, dbuf_a, ubuf_b, dbuf_b,
                  acc, usem_a, dsem_a, usem_b, dsem_b):
    w = _wid()

    def u_slice(c):
        return u_hbm.at[pl.ds(c * SCHUNK, SCHUNK)]

    def d_slice(col, c):
        return mt_hbm.at[col, pl.ds(c * SCHUNK, SCHUNK)]

    def start(col, c, ub, db, us, ds_):
        pltpu.async_copy(u_slice(c), ub, us)
        pltpu.async_copy(d_slice(col, c), db, ds_)

    def wait(col, c, ub, db, us, ds_):
        pltpu.make_async_copy(u_slice(c), ub, us).wait()
        pltpu.make_async_copy(d_slice(col, c), db, ds_).wait()

    def compute(ubuf, dbuf):
        def grp(g, mx_c):
            pk = ubuf[pl.ds(g * 16, 16)]
            d16 = dbuf[pl.ds(g * 16, 16)]
            u16 = lax.bitwise_and(pk, 0xFFFFF)
            od = lax.shift_right_logical(pk, 20)
            plsc.addupdate_scatter(acc, [u16], d16, mask=od == 0)
            return jnp.maximum(mx_c, od)

        mx = plsc.parallel_loop(0, SCHUNK // 16, unroll=16,
                                carry=jnp.zeros((16,), I32))(grp)
        mmax = jnp.max(mx)

        def extra(r, carry3):
            def grp2(g, c3):
                pk = ubuf[pl.ds(g * 16, 16)]
                d16 = dbuf[pl.ds(g * 16, 16)]
                u16 = lax.bitwise_and(pk, 0xFFFFF)
                od = lax.shift_right_logical(pk, 20)
                plsc.addupdate_scatter(acc, [u16], d16, mask=od == r)
                return c3

            return lax.fori_loop(0, SCHUNK // 16, grp2, carry3)

        lax.fori_loop(1, mmax + 1, extra, 0)

    def col_round(ri, carry):
        col = w + 32 * ri

        def zstep(i):
            acc[pl.ds(i * 16, 16)] = jnp.zeros((16,), F32)

        plsc.parallel_loop(0, NPAD // 16, unroll=16)(zstep)

        start(col, 0, ubuf_a, dbuf_a, usem_a, dsem_a)

        def pair(t, carry2):
            c0 = 2 * t
            c1 = 2 * t + 1
            wait(col, c0, ubuf_a, dbuf_a, usem_a, dsem_a)
            start(col, c1, ubuf_b, dbuf_b, usem_b, dsem_b)
            compute(ubuf_a, dbuf_a)
            wait(col, c1, ubuf_b, dbuf_b, usem_b, dsem_b)

            @pl.when(c1 + 1 < NCHUNK)
            def _():
                start(col, c1 + 1, ubuf_a, dbuf_a, usem_a, dsem_a)

            compute(ubuf_b, dbuf_b)
            return carry2

        lax.fori_loop(0, NCHUNK // 2, pair, 0)
        pltpu.sync_copy(acc, agg_hbm.at[col])
        return carry

    lax.fori_loop(0, 4, col_round, 0)


_scatter = pl.kernel(
    _scatter_body,
    out_type=jax.ShapeDtypeStruct((D, NPAD), F32),
    mesh=_mesh,
    compiler_params=pltpu.CompilerParams(needs_layout_passes=False),
    scratch_types=[
        pltpu.VMEM((SCHUNK,), I32),
        pltpu.VMEM((SCHUNK,), F32),
        pltpu.VMEM((SCHUNK,), I32),
        pltpu.VMEM((SCHUNK,), F32),
        pltpu.VMEM((NPAD,), F32),
        pltpu.SemaphoreType.DMA,
        pltpu.SemaphoreType.DMA,
        pltpu.SemaphoreType.DMA,
        pltpu.SemaphoreType.DMA,
    ],
)


def _k2_body(x_ref, w_ref, o_ref):
    b = pl.program_id(0)
    x = x_ref[...].astype(jnp.bfloat16)
    w = w_ref[0].astype(jnp.bfloat16)
    y = lax.dot_general(w, x, (((0,), (1,)), ((), ())),
                        preferred_element_type=F32)
    rel_j = lax.rem(b, 98)
    valid = jnp.where(b >= 7 * 98, 0, jnp.where(rel_j == 97, 336, 512))
    cols = lax.broadcasted_iota(I32, (D, 512), 1)
    o_ref[...] = jnp.where(cols < valid, y, 0.0)


def _k2_call(g, w7):
    nb = EH // 512
    return pl.pallas_call(
        _k2_body,
        grid=(nb,),
        in_specs=[
            pl.BlockSpec((512, D), lambda b: (b, 0)),
            pl.BlockSpec((1, D, D), lambda b: (jnp.minimum(b // 98, 6), 0, 0)),
        ],
        out_specs=pl.BlockSpec((D, 512), lambda b: (0, b)),
        out_shape=jax.ShapeDtypeStruct((D, EH), F32),
    )(g, w7)


def _k0_body(c_ref, f_ref, iw1, ib1, iw2, ib2, sw1, sb1, sw2, sb2, o_ref, ob_ref):
    h = jax.nn.relu(jnp.dot(c_ref[...], iw1[...], preferred_element_type=F32) + ib1[...])
    a = jax.nn.relu(jnp.dot(h, iw2[...], preferred_element_type=F32) + ib2[...])
    h = jax.nn.relu(jnp.dot(f_ref[...], sw1[...], preferred_element_type=F32) + sb1[...])
    b2 = jax.nn.relu(jnp.dot(h, sw2[...], preferred_element_type=F32) + sb2[...])
    out = jax.nn.relu(a + b2)
    o_ref[...] = out
    ob_ref[...] = out.astype(jnp.bfloat16)


def _k0_call(ctrs_p, feats_p, p):
    full = lambda shp: pl.BlockSpec(shp, lambda j: tuple(0 for _ in shp))
    return pl.pallas_call(
        _k0_body,
        grid=(NPAD // 1024,),
        in_specs=[
            pl.BlockSpec((1024, 2), lambda j: (j, 0)),
            pl.BlockSpec((1024, 2), lambda j: (j, 0)),
            full((2, D)), full((1, D)), full((D, D)), full((1, D)),
            full((2, D)), full((1, D)), full((D, D)), full((1, D)),
        ],
        out_specs=[pl.BlockSpec((1024, D), lambda j: (j, 0)),
                   pl.BlockSpec((1024, D), lambda j: (j, 0))],
        out_shape=[jax.ShapeDtypeStruct((NPAD, D), F32),
                   jax.ShapeDtypeStruct((NPAD, D), jnp.bfloat16)],
    )(ctrs_p, feats_p,
      p["in_W1"], p["in_b1"].reshape(1, D), p["in_W2"], p["in_b2"].reshape(1, D),
      p["seg_W1"], p["seg_b1"].reshape(1, D), p["seg_W2"], p["seg_b2"].reshape(1, D))


def _k4_body(f_ref, at_ref, at2_ref, cw, g_ref, b_ref, w1, b1, w2, b2, o_ref, ob_ref):
    feat = f_ref[...]
    ats = at_ref[...] + at2_ref[...]
    parts = [jnp.transpose(ats[:, t * 128:(t + 1) * 128]) for t in range(8)]
    agg = jnp.concatenate(parts, axis=0)
    temp = jnp.dot(feat, cw[...], preferred_element_type=F32) + agg
    mu = jnp.mean(temp, axis=-1, keepdims=True)
    var = jnp.mean((temp - mu) ** 2, axis=-1, keepdims=True)
    x = (temp - mu) / jnp.sqrt(var + 1e-5) * g_ref[...] + b_ref[...]
    x = jax.nn.relu(x)
    h = jax.nn.relu(jnp.dot(x, w1[...], preferred_element_type=F32) + b1[...])
    o = jax.nn.relu(jnp.dot(h, w2[...], preferred_element_type=F32) + b2[...])
    out = jax.nn.relu(o + feat)
    o_ref[...] = out
    ob_ref[...] = out.astype(jnp.bfloat16)


def _k4_call(feat, agg_t, agg_t2, cw, g, b, w1, b1, w2, b2):
    full = lambda shp: pl.BlockSpec(shp, lambda j: tuple(0 for _ in shp))
    return pl.pallas_call(
        _k4_body,
        grid=(NPAD // 1024,),
        in_specs=[
            pl.BlockSpec((1024, D), lambda j: (j, 0)),
            pl.BlockSpec((D, 1024), lambda j: (0, j)),
            pl.BlockSpec((D, 1024), lambda j: (0, j)),
            full((D, D)), full((1, D)), full((1, D)),
            full((D, D)), full((1, D)), full((D, D)), full((1, D)),
        ],
        out_specs=[pl.BlockSpec((1024, D), lambda j: (j, 0)),
                   pl.BlockSpec((1024, D), lambda j: (j, 0))],
        out_shape=[jax.ShapeDtypeStruct((NPAD, D), F32),
                   jax.ShapeDtypeStruct((NPAD, D), jnp.bfloat16)],
    )(feat, agg_t, agg_t2, cw, g.reshape(1, D), b.reshape(1, D),
      w1, b1.reshape(1, D), w2, b2.reshape(1, D))


def _k5_body(f_ref, m_ref, emb, wf, wm, b1, w2, b2, o_ref):
    m = m_ref[0, 0, :]
    iota = lax.broadcasted_iota(I32, (1000, 32), 1)
    oh = (m[:, None] == iota).astype(F32)
    meta = jnp.dot(oh, emb[...], preferred_element_type=F32)
    h = jax.nn.relu(jnp.dot(f_ref[...], wf[...], preferred_element_type=F32)
                    + jnp.dot(meta, wm[...], preferred_element_type=F32) + b1[...])
    o_ref[...] = jax.nn.relu(jnp.dot(h, w2[...], preferred_element_type=F32) + b2[...])


def _k5_call(feat, mark3d, emb_p, wf, wm, b1, w2, b2):
    full = lambda shp: pl.BlockSpec(shp, lambda j: tuple(0 for _ in shp))
    return pl.pallas_call(
        _k5_body,
        grid=(N // 1000,),
        in_specs=[
            pl.BlockSpec((1000, D), lambda j: (j, 0)),
            pl.BlockSpec((1, 1, 1000), lambda j: (j, 0, 0)),
            full((32, D)), full((D, D)), full((D, D)), full((1, D)),
            full((D, D)), full((1, D)),
        ],
        out_specs=pl.BlockSpec((1000, D), lambda j: (j, 0)),
        out_shape=jax.ShapeDtypeStruct((N, D), F32),
    )(feat, mark3d, emb_p, wf, wm, b1.reshape(1, D), w2, b2.reshape(1, D))


def kernel(ctrs, feats, mark_type, idcs, pre_u, pre_v, suc_u, suc_v,
           left_u, left_v, right_u, right_v, params):
    p = params
    depth = p["ctr_W"].shape[0]

    def pad_rel(a):
        pad = (jnp.arange(EPAD - E, dtype=I32) * 571) % N
        return jnp.concatenate([a.astype(I32), pad])

    def pad_rel_v(a):
        return jnp.concatenate([a.astype(I32), jnp.zeros((EPAD - E,), I32)])

    extra_u = (jnp.arange(1024, dtype=I32) * 49) % N
    u_parts = [pad_rel(pre_u[s]) for s in range(6)]
    u_parts += [pad_rel(suc_u[0]), extra_u]
    u_parts += [pad_rel(suc_u[s]) for s in range(1, 6)]
    u_parts += [pad_rel(left_u), pad_rel(right_u), extra_u]
    u_all = jnp.concatenate(u_parts)
    u2 = u_all.reshape(-1, 16)
    eq = u2[:, :, None] == u2[:, None, :]
    tril = (jnp.arange(16)[:, None] > jnp.arange(16)[None, :])
    ordi = jnp.sum(eq & tril[None], axis=-1, dtype=I32)
    u_all = (u_all | (ordi.reshape(-1) << 20)).reshape(2, EH)

    extra_v = jnp.zeros((1024,), I32)
    v_parts = [pad_rel_v(pre_v[s]) for s in range(6)]
    v_parts += [pad_rel_v(suc_v[0]), extra_v]
    v_parts += [pad_rel_v(suc_v[s]) for s in range(1, 6)]
    v_parts += [pad_rel_v(left_v), pad_rel_v(right_v), extra_v]
    v_all = jnp.concatenate(v_parts).reshape(2, 32, GB, 128)

    w_all = jnp.concatenate(
        [p["pre_W"], p["suc_W"], p["left_W"][:, None], p["right_W"][:, None]],
        axis=1)

    ctrs_p = jnp.pad(ctrs, ((0, NPAD - N), (0, 0)))
    feats_p = jnp.pad(feats, ((0, NPAD - N), (0, 0)))
    emb_p = jnp.pad(p["emb"], ((0, 32 - p["emb"].shape[0]), (0, 0)))
    mark3d = mark_type.astype(I32).reshape(N // 1000, 1, 1000)
    wf = p["meta_W1"][:D]
    wm = p["meta_W1"][D:]

    feat, featb = _k0_call(ctrs_p, feats_p, p)
    del featb
    for i in range(depth):
        g0 = _gather(feat, v_all[0])
        m0 = _k2_call(g0, w_all[i, :7])
        g1 = _gather(feat, v_all[1])
        a0 = _scatter(m0, u_all[0])
        m1 = _k2_call(g1, w_all[i, 7:])
        a1 = _scatter(m1, u_all[1])
        feat, _fb = _k4_call(feat, a0, a1, p["ctr_W"][i], p["gn_g"][i],
                             p["gn_b"][i], p["ctr2_W1"][i], p["ctr2_b1"][i],
                             p["ctr2_W2"][i], p["ctr2_b2"][i])
    out = _k5_call(feat, mark3d, emb_p, wf, wm,
                   p["meta_b1"], p["meta_W2"], p["meta_b2"])
    return (out, idcs, ctrs)
